# baseline (speedup 1.0000x reference)
import jax
import jax.numpy as jnp
from jax.experimental import pallas as pl
from jax.experimental.pallas import tpu as pltpu

_N = 50000
_G = 16
_C = 1061
_CH = 1000
_NC = _N // _CH
_DEPTH = 5
_NR = _NC // _DEPTH


def _head_kernel(x_hbm, bt_hbm, first_ref, last_ref,
                 w0_ref, b0_ref, w1_ref, b1_ref, w2_ref, b2_ref,
                 w3_ref, b3_ref, w4_ref, b4_ref,
                 wv_ref, bv_ref, wo_ref, bo_ref,
                 c1w_ref, c1b_ref, c2w_ref, c2b_ref,
                 o_ref, buf_ref, sem_ref, btb_ref, btsem_ref,
                 acc_ref, cnt_ref):
    acc_ref[...] = jnp.zeros_like(acc_ref)
    cnt_ref[...] = jnp.zeros_like(cnt_ref)

    def chunk_copy(c, slot):
        return pltpu.make_async_copy(
            x_hbm.at[pl.ds(c * _CH, _CH), :],
            buf_ref.at[slot],
            sem_ref.at[slot])

    for d in range(_DEPTH):
        chunk_copy(d, d).start()

    def round_body(r, carry):
        base = r * _DEPTH
        for d in range(_DEPTH):
            c = base + d
            chunk_copy(c, d).wait()

            g_first = first_ref[c]
            uniform = g_first == last_ref[c]

            @pl.when(uniform)
            def _interior(d=d, g_first=g_first):
                colsum = jnp.sum(buf_ref[d], axis=0, keepdims=True)
                sel = (jax.lax.broadcasted_iota(jnp.int32, (_G, 1), 0)
                       == g_first).astype(jnp.float32)
                acc_ref[...] += sel * colsum
                cnt_ref[...] += sel * float(_CH)

            @pl.when(jnp.logical_not(uniform))
            def _boundary(c=c, d=d):
                bt_dma = pltpu.make_async_copy(
                    bt_hbm.at[pl.ds(c * _CH, _CH), :], btb_ref, btsem_ref)
                bt_dma.start()
                bt_dma.wait()
                bts = btb_ref[...]
                onehot = (bts == jax.lax.broadcasted_iota(
                    jnp.int32, (1, _G), 1)).astype(jnp.float32)
                acc_ref[...] += jax.lax.dot_general(
                    onehot, buf_ref[d], (((0,), (0,)), ((), ())),
                    preferred_element_type=jnp.float32,
                    precision=jax.lax.Precision.HIGHEST)
                cnt_ref[...] += jnp.sum(onehot, axis=0)[:, None]

            @pl.when(c + _DEPTH < _NC)
            def _prefetch(c=c, d=d):
                chunk_copy(c + _DEPTH, d).start()
        return carry

    jax.lax.fori_loop(0, _NR, round_body, 0)

    c = jnp.maximum(cnt_ref[...], 1.0)
    hb = acc_ref[...][:, 37:] / c

    def lin(h, w_ref, b_ref, relu):
        y = jax.lax.dot_general(
            h, w_ref[...], (((1,), (0,)), ((), ())),
            preferred_element_type=jnp.float32,
            precision=jax.lax.Precision.HIGHEST) + b_ref[...]
        return jnp.maximum(y, 0.0) if relu else y

    hb = lin(hb, w0_ref, b0_ref, True)
    hb = lin(hb, w1_ref, b1_ref, True)
    hb = lin(hb, w2_ref, b2_ref, True)
    hb = lin(hb, w3_ref, b3_ref, True)
    hb = lin(hb, w4_ref, b4_ref, True)
    fused = lin(lin(hb, wv_ref, bv_ref, False), wo_ref, bo_ref, False)
    z = lin(fused, c1w_ref, c1b_ref, True)
    o_ref[...] = lin(z, c2w_ref, c2b_ref, False)


def kernel(x, edge_index, batch, params):
    del edge_index
    bt2d = batch.reshape(_N, 1)
    btc = batch.reshape(_NC, _CH)
    first_ids = btc[:, 0]
    last_ids = btc[:, _CH - 1]

    def wspec():
        return pl.BlockSpec(memory_space=pltpu.MemorySpace.VMEM)

    weights = []
    wspecs = []
    for nm in ['sp_l0', 'sp_l1', 'sp_l2', 'sp_l3', 'sp_l4']:
        w = params[nm + '_w']
        b = params[nm + '_b'].reshape(1, -1)
        weights += [w, b]
        wspecs += [wspec(), wspec()]
    for nm in ['mha_wv', 'mha_bv', 'mha_wo', 'mha_bo',
               'cls_l1_w', 'cls_l1_b', 'cls_l2_w', 'cls_l2_b']:
        a = params[nm]
        if a.ndim == 1:
            a = a.reshape(1, -1)
        weights.append(a)
        wspecs.append(wspec())

    return pl.pallas_call(
        _head_kernel,
        in_specs=[
            pl.BlockSpec(memory_space=pltpu.MemorySpace.HBM),
            pl.BlockSpec(memory_space=pltpu.MemorySpace.HBM),
            pl.BlockSpec(memory_space=pltpu.MemorySpace.SMEM),
            pl.BlockSpec(memory_space=pltpu.MemorySpace.SMEM),
        ] + wspecs,
        out_specs=pl.BlockSpec(memory_space=pltpu.MemorySpace.VMEM),
        out_shape=jax.ShapeDtypeStruct((_G, 2), jnp.float32),
        scratch_shapes=[
            pltpu.VMEM((_DEPTH, _CH, _C), jnp.float32),
            pltpu.SemaphoreType.DMA((_DEPTH,)),
            pltpu.VMEM((_CH, 1), jnp.int32),
            pltpu.SemaphoreType.DMA,
            pltpu.VMEM((_G, _C), jnp.float32),
            pltpu.VMEM((_G, 1), jnp.float32),
        ],
    )(x, bt2d, first_ids, last_ids, *weights)


# speedup vs baseline: 1.1295x; 1.1295x over previous
import jax
import jax.numpy as jnp
from jax.experimental import pallas as pl
from jax.experimental.pallas import tpu as pltpu

_N = 50000
_G = 16
_C = 1061
_CH = 1000
_NC = _N // _CH
_DEPTH = 5
_NR = _NC // _DEPTH


def _head_kernel(x_hbm, bt_ref, first_ref, last_ref,
                 w0_ref, b0_ref, w1_ref, b1_ref, w2_ref, b2_ref,
                 w3_ref, b3_ref, w4_ref, b4_ref,
                 wv_ref, bv_ref, wo_ref, bo_ref,
                 c1w_ref, c1b_ref, c2w_ref, c2b_ref,
                 o_ref, buf_ref, sem_ref, acc_ref, cnt_ref):
    acc_ref[...] = jnp.zeros_like(acc_ref)
    cnt_ref[...] = jnp.zeros_like(cnt_ref)

    def chunk_copy(c, slot):
        return pltpu.make_async_copy(
            x_hbm.at[pl.ds(c * _CH, _CH), :],
            buf_ref.at[slot],
            sem_ref.at[slot])

    for d in range(_DEPTH):
        chunk_copy(d, d).start()

    def round_body(r, carry):
        base = r * _DEPTH
        for d in range(_DEPTH):
            c = base + d
            chunk_copy(c, d).wait()

            g_first = first_ref[c]
            uniform = g_first == last_ref[c]

            @pl.when(uniform)
            def _interior(d=d, g_first=g_first):
                colsum = jnp.sum(buf_ref[d], axis=0, keepdims=True)
                sel = (jax.lax.broadcasted_iota(jnp.int32, (_G, 1), 0)
                       == g_first).astype(jnp.float32)
                acc_ref[...] += sel * colsum
                cnt_ref[...] += sel * float(_CH)

            @pl.when(jnp.logical_not(uniform))
            def _boundary(c=c, d=d):
                bts = bt_ref[pl.ds(c * _CH, _CH), :]
                onehot = (bts == jax.lax.broadcasted_iota(
                    jnp.int32, (1, _G), 1)).astype(jnp.float32)
                acc_ref[...] += jax.lax.dot_general(
                    onehot, buf_ref[d], (((0,), (0,)), ((), ())),
                    preferred_element_type=jnp.float32,
                    precision=jax.lax.Precision.HIGHEST)
                cnt_ref[...] += jnp.sum(onehot, axis=0)[:, None]

            @pl.when(c + _DEPTH < _NC)
            def _prefetch(c=c, d=d):
                chunk_copy(c + _DEPTH, d).start()
        return carry

    jax.lax.fori_loop(0, _NR, round_body, 0)

    c = jnp.maximum(cnt_ref[...], 1.0)
    hb = acc_ref[...][:, 37:] / c

    def lin(h, w_ref, b_ref, relu):
        y = jax.lax.dot_general(
            h, w_ref[...], (((1,), (0,)), ((), ())),
            preferred_element_type=jnp.float32,
            precision=jax.lax.Precision.HIGHEST) + b_ref[...]
        return jnp.maximum(y, 0.0) if relu else y

    hb = lin(hb, w0_ref, b0_ref, True)
    hb = lin(hb, w1_ref, b1_ref, True)
    hb = lin(hb, w2_ref, b2_ref, True)
    hb = lin(hb, w3_ref, b3_ref, True)
    hb = lin(hb, w4_ref, b4_ref, True)
    fused = lin(lin(hb, wv_ref, bv_ref, False), wo_ref, bo_ref, False)
    z = lin(fused, c1w_ref, c1b_ref, True)
    o_ref[...] = lin(z, c2w_ref, c2b_ref, False)


def kernel(x, edge_index, batch, params):
    del edge_index
    bt2d = batch.reshape(_N, 1)
    btc = batch.reshape(_NC, _CH)
    first_ids = btc[:, 0]
    last_ids = btc[:, _CH - 1]

    def wspec():
        return pl.BlockSpec(memory_space=pltpu.MemorySpace.VMEM)

    weights = []
    wspecs = []
    for nm in ['sp_l0', 'sp_l1', 'sp_l2', 'sp_l3', 'sp_l4']:
        w = params[nm + '_w']
        b = params[nm + '_b'].reshape(1, -1)
        weights += [w, b]
        wspecs += [wspec(), wspec()]
    for nm in ['mha_wv', 'mha_bv', 'mha_wo', 'mha_bo',
               'cls_l1_w', 'cls_l1_b', 'cls_l2_w', 'cls_l2_b']:
        a = params[nm]
        if a.ndim == 1:
            a = a.reshape(1, -1)
        weights.append(a)
        wspecs.append(wspec())

    return pl.pallas_call(
        _head_kernel,
        in_specs=[
            pl.BlockSpec(memory_space=pltpu.MemorySpace.HBM),
            pl.BlockSpec(memory_space=pltpu.MemorySpace.VMEM),
            pl.BlockSpec(memory_space=pltpu.MemorySpace.SMEM),
            pl.BlockSpec(memory_space=pltpu.MemorySpace.SMEM),
        ] + wspecs,
        out_specs=pl.BlockSpec(memory_space=pltpu.MemorySpace.VMEM),
        out_shape=jax.ShapeDtypeStruct((_G, 2), jnp.float32),
        scratch_shapes=[
            pltpu.VMEM((_DEPTH, _CH, _C), jnp.float32),
            pltpu.SemaphoreType.DMA((_DEPTH,)),
            pltpu.VMEM((_G, _C), jnp.float32),
            pltpu.VMEM((_G, 1), jnp.float32),
        ],
    )(x, bt2d, first_ids, last_ids, *weights)


# speedup vs baseline: 1.1461x; 1.0146x over previous
import jax
import jax.numpy as jnp
from jax.experimental import pallas as pl
from jax.experimental.pallas import tpu as pltpu

_N = 50000
_G = 16
_C = 1061
_CH = 1000
_NC = _N // _CH
_DEPTH = 5
_NR = _NC // _DEPTH


def _head_kernel(x_hbm, lo_ref, hi_ref, first_ref, last_ref,
                 w0_ref, b0_ref, w1_ref, b1_ref, w2_ref, b2_ref,
                 w3_ref, b3_ref, w4_ref, b4_ref,
                 wv_ref, bv_ref, wo_ref, bo_ref,
                 c1w_ref, c1b_ref, c2w_ref, c2b_ref,
                 o_ref, buf_ref, sem_ref, acc_ref, cnt_ref):
    acc_ref[...] = jnp.zeros_like(acc_ref)
    cnt_ref[...] = jnp.zeros_like(cnt_ref)

    def chunk_copy(c, slot):
        return pltpu.make_async_copy(
            x_hbm.at[pl.ds(c * _CH, _CH), :],
            buf_ref.at[slot],
            sem_ref.at[slot])

    for d in range(_DEPTH):
        chunk_copy(d, d).start()

    def round_body(r, carry):
        base = r * _DEPTH
        for d in range(_DEPTH):
            c = base + d
            chunk_copy(c, d).wait()

            g_first = first_ref[c]
            uniform = g_first == last_ref[c]

            @pl.when(uniform)
            def _interior(d=d, g_first=g_first):
                colsum = jnp.sum(buf_ref[d], axis=0, keepdims=True)
                sel = (jax.lax.broadcasted_iota(jnp.int32, (_G, 1), 0)
                       == g_first).astype(jnp.float32)
                acc_ref[...] += sel * colsum
                cnt_ref[...] += sel * float(_CH)

            @pl.when(jnp.logical_not(uniform))
            def _boundary(c=c, d=d):
                rowid = jax.lax.broadcasted_iota(
                    jnp.int32, (_CH, _G), 0) + c * _CH
                onehot = jnp.logical_and(
                    rowid >= lo_ref[...], rowid < hi_ref[...]
                ).astype(jnp.float32)
                acc_ref[...] += jax.lax.dot_general(
                    onehot, buf_ref[d], (((0,), (0,)), ((), ())),
                    preferred_element_type=jnp.float32,
                    precision=jax.lax.Precision.HIGHEST)
                cnt_ref[...] += jnp.sum(onehot, axis=0)[:, None]

            @pl.when(c + _DEPTH < _NC)
            def _prefetch(c=c, d=d):
                chunk_copy(c + _DEPTH, d).start()
        return carry

    jax.lax.fori_loop(0, _NR, round_body, 0)

    c = jnp.maximum(cnt_ref[...], 1.0)
    hb = acc_ref[...][:, 37:] / c

    def lin(h, w_ref, b_ref, relu):
        y = jax.lax.dot_general(
            h, w_ref[...], (((1,), (0,)), ((), ())),
            preferred_element_type=jnp.float32,
            precision=jax.lax.Precision.HIGHEST) + b_ref[...]
        return jnp.maximum(y, 0.0) if relu else y

    hb = lin(hb, w0_ref, b0_ref, True)
    hb = lin(hb, w1_ref, b1_ref, True)
    hb = lin(hb, w2_ref, b2_ref, True)
    hb = lin(hb, w3_ref, b3_ref, True)
    hb = lin(hb, w4_ref, b4_ref, True)
    fused = lin(lin(hb, wv_ref, bv_ref, False), wo_ref, bo_ref, False)
    z = lin(fused, c1w_ref, c1b_ref, True)
    o_ref[...] = lin(z, c2w_ref, c2b_ref, False)


def kernel(x, edge_index, batch, params):
    del edge_index
    btc = batch.reshape(_NC, _CH)
    first_ids = btc[:, 0]
    last_ids = btc[:, _CH - 1]
    starts = jnp.searchsorted(
        batch, jnp.arange(_G + 1, dtype=batch.dtype)).astype(jnp.int32)
    starts_lo = starts[:_G].reshape(1, _G)
    starts_hi = starts[1:].reshape(1, _G)

    def wspec():
        return pl.BlockSpec(memory_space=pltpu.MemorySpace.VMEM)

    weights = []
    wspecs = []
    for nm in ['sp_l0', 'sp_l1', 'sp_l2', 'sp_l3', 'sp_l4']:
        w = params[nm + '_w']
        b = params[nm + '_b'].reshape(1, -1)
        weights += [w, b]
        wspecs += [wspec(), wspec()]
    for nm in ['mha_wv', 'mha_bv', 'mha_wo', 'mha_bo',
               'cls_l1_w', 'cls_l1_b', 'cls_l2_w', 'cls_l2_b']:
        a = params[nm]
        if a.ndim == 1:
            a = a.reshape(1, -1)
        weights.append(a)
        wspecs.append(wspec())

    return pl.pallas_call(
        _head_kernel,
        in_specs=[
            pl.BlockSpec(memory_space=pltpu.MemorySpace.HBM),
            pl.BlockSpec(memory_space=pltpu.MemorySpace.VMEM),
            pl.BlockSpec(memory_space=pltpu.MemorySpace.VMEM),
            pl.BlockSpec(memory_space=pltpu.MemorySpace.SMEM),
            pl.BlockSpec(memory_space=pltpu.MemorySpace.SMEM),
        ] + wspecs,
        out_specs=pl.BlockSpec(memory_space=pltpu.MemorySpace.VMEM),
        out_shape=jax.ShapeDtypeStruct((_G, 2), jnp.float32),
        scratch_shapes=[
            pltpu.VMEM((_DEPTH, _CH, _C), jnp.float32),
            pltpu.SemaphoreType.DMA((_DEPTH,)),
            pltpu.VMEM((_G, _C), jnp.float32),
            pltpu.VMEM((_G, 1), jnp.float32),
        ],
    )(x, starts_lo, starts_hi, first_ids, last_ids, *weights)


# speedup vs baseline: 1.1551x; 1.0079x over previous
import jax
import jax.numpy as jnp
from jax.experimental import pallas as pl
from jax.experimental.pallas import tpu as pltpu

_N = 50000
_G = 16
_C = 1061
_CH = 1000
_NC = _N // _CH
_DEPTH = 5
_NR = _NC // _DEPTH


def _head_kernel(x_hbm, lo_ref, hi_ref, first_ref, last_ref,
                 w0_ref, b0_ref, w1_ref, b1_ref, w2_ref, b2_ref,
                 w3_ref, b3_ref, w4_ref, b4_ref,
                 wv_ref, bv_ref, wo_ref, bo_ref,
                 c1w_ref, c1b_ref, c2w_ref, c2b_ref,
                 o_ref, buf_ref, sem_ref, semb_ref, acc_ref, cnt_ref):
    acc_ref[...] = jnp.zeros_like(acc_ref)
    cnt_ref[...] = jnp.zeros_like(cnt_ref)
    _H = 504

    def chunk_copies(c, slot):
        return (
            pltpu.make_async_copy(
                x_hbm.at[pl.ds(c * _CH, _H), :],
                buf_ref.at[slot, pl.ds(0, _H), :],
                sem_ref.at[slot]),
            pltpu.make_async_copy(
                x_hbm.at[pl.ds(c * _CH + _H, _CH - _H), :],
                buf_ref.at[slot, pl.ds(_H, _CH - _H), :],
                semb_ref.at[slot]),
        )

    def start_chunk(c, slot):
        a, b = chunk_copies(c, slot)
        a.start()
        b.start()

    def wait_chunk(c, slot):
        a, b = chunk_copies(c, slot)
        a.wait()
        b.wait()

    for d in range(_DEPTH):
        start_chunk(d, d)

    def round_body(r, carry):
        base = r * _DEPTH
        for d in range(_DEPTH):
            c = base + d
            wait_chunk(c, d)

            g_first = first_ref[c]
            uniform = g_first == last_ref[c]

            @pl.when(uniform)
            def _interior(d=d, g_first=g_first):
                colsum = jnp.sum(buf_ref[d], axis=0, keepdims=True)
                sel = (jax.lax.broadcasted_iota(jnp.int32, (_G, 1), 0)
                       == g_first).astype(jnp.float32)
                acc_ref[...] += sel * colsum
                cnt_ref[...] += sel * float(_CH)

            @pl.when(jnp.logical_not(uniform))
            def _boundary(c=c, d=d):
                rowid = jax.lax.broadcasted_iota(
                    jnp.int32, (_CH, _G), 0) + c * _CH
                onehot = jnp.logical_and(
                    rowid >= lo_ref[...], rowid < hi_ref[...]
                ).astype(jnp.float32)
                acc_ref[...] += jax.lax.dot_general(
                    onehot, buf_ref[d], (((0,), (0,)), ((), ())),
                    preferred_element_type=jnp.float32,
                    precision=jax.lax.Precision.HIGHEST)
                cnt_ref[...] += jnp.sum(onehot, axis=0)[:, None]

            @pl.when(c + _DEPTH < _NC)
            def _prefetch(c=c, d=d):
                start_chunk(c + _DEPTH, d)
        return carry

    jax.lax.fori_loop(0, _NR, round_body, 0)

    c = jnp.maximum(cnt_ref[...], 1.0)
    hb = acc_ref[...][:, 37:] / c

    def lin(h, w_ref, b_ref, relu):
        y = jax.lax.dot_general(
            h, w_ref[...], (((1,), (0,)), ((), ())),
            preferred_element_type=jnp.float32,
            precision=jax.lax.Precision.HIGHEST) + b_ref[...]
        return jnp.maximum(y, 0.0) if relu else y

    hb = lin(hb, w0_ref, b0_ref, True)
    hb = lin(hb, w1_ref, b1_ref, True)
    hb = lin(hb, w2_ref, b2_ref, True)
    hb = lin(hb, w3_ref, b3_ref, True)
    hb = lin(hb, w4_ref, b4_ref, True)
    fused = lin(lin(hb, wv_ref, bv_ref, False), wo_ref, bo_ref, False)
    z = lin(fused, c1w_ref, c1b_ref, True)
    o_ref[...] = lin(z, c2w_ref, c2b_ref, False)


def kernel(x, edge_index, batch, params):
    del edge_index
    btc = batch.reshape(_NC, _CH)
    first_ids = btc[:, 0]
    last_ids = btc[:, _CH - 1]
    starts = jnp.searchsorted(
        batch, jnp.arange(_G + 1, dtype=batch.dtype)).astype(jnp.int32)
    starts_lo = starts[:_G].reshape(1, _G)
    starts_hi = starts[1:].reshape(1, _G)

    def wspec():
        return pl.BlockSpec(memory_space=pltpu.MemorySpace.VMEM)

    weights = []
    wspecs = []
    for nm in ['sp_l0', 'sp_l1', 'sp_l2', 'sp_l3', 'sp_l4']:
        w = params[nm + '_w']
        b = params[nm + '_b'].reshape(1, -1)
        weights += [w, b]
        wspecs += [wspec(), wspec()]
    for nm in ['mha_wv', 'mha_bv', 'mha_wo', 'mha_bo',
               'cls_l1_w', 'cls_l1_b', 'cls_l2_w', 'cls_l2_b']:
        a = params[nm]
        if a.ndim == 1:
            a = a.reshape(1, -1)
        weights.append(a)
        wspecs.append(wspec())

    return pl.pallas_call(
        _head_kernel,
        in_specs=[
            pl.BlockSpec(memory_space=pltpu.MemorySpace.HBM),
            pl.BlockSpec(memory_space=pltpu.MemorySpace.VMEM),
            pl.BlockSpec(memory_space=pltpu.MemorySpace.VMEM),
            pl.BlockSpec(memory_space=pltpu.MemorySpace.SMEM),
            pl.BlockSpec(memory_space=pltpu.MemorySpace.SMEM),
        ] + wspecs,
        out_specs=pl.BlockSpec(memory_space=pltpu.MemorySpace.VMEM),
        out_shape=jax.ShapeDtypeStruct((_G, 2), jnp.float32),
        scratch_shapes=[
            pltpu.VMEM((_DEPTH, _CH, _C), jnp.float32),
            pltpu.SemaphoreType.DMA((_DEPTH,)),
            pltpu.SemaphoreType.DMA((_DEPTH,)),
            pltpu.VMEM((_G, _C), jnp.float32),
            pltpu.VMEM((_G, 1), jnp.float32),
        ],
    )(x, starts_lo, starts_hi, first_ids, last_ids, *weights)
